# baseline (speedup 1.0000x reference)
import functools

import jax
import jax.numpy as jnp
from jax.experimental import pallas as pl
from jax.experimental.pallas import tpu as pltpu

TOP_K = 8
N_EXPERTS = 64
N_GROUP = 8
GROUP_SIZE = N_EXPERTS // N_GROUP
TOPK_GROUP = 4
SCALE = 2.5

TILE = 2048


def _gate_kernel(x_ref, w_ref, b_ref, idx_ref, wgt_ref):
    x = x_ref[...]
    w = w_ref[...]
    logits = jax.lax.dot_general(
        x, w, (((1,), (1,)), ((), ())),
        preferred_element_type=jnp.float32,
    )
    s = jax.nn.sigmoid(logits)
    sc = s + b_ref[...]

    t = x.shape[0]
    lane = jax.lax.broadcasted_iota(jnp.int32, (t, N_EXPERTS), 1)
    lane_f = lane.astype(jnp.float32)
    gid = lane // GROUP_SIZE
    lanem = lane % GROUP_SIZE
    neg = jnp.float32(-jnp.inf)

    a = sc
    b = jnp.full_like(sc, neg)
    for d in (1, 2, 4):
        ar = jnp.roll(a, -d, axis=1)
        br = jnp.roll(b, -d, axis=1)
        valid = lanem < (GROUP_SIZE - d)
        ar = jnp.where(valid, ar, neg)
        br = jnp.where(valid, br, neg)
        b = jnp.maximum(jnp.maximum(b, br), jnp.minimum(a, ar))
        a = jnp.maximum(a, ar)
    g = a + b
    for d in (1, 2, 4):
        g = jnp.where(lanem >= d, jnp.roll(g, d, axis=1), g)

    rank = jnp.zeros_like(lane)
    for k in range(1, N_GROUP):
        other = jnp.roll(g, -GROUP_SIZE * k, axis=1)
        og_lt = gid >= (N_GROUP - k)
        beats = (other > g) | ((other == g) & og_lt)
        rank = rank + beats.astype(jnp.int32)
    tmp = jnp.where(rank < TOPK_GROUP, sc, neg)

    idx_cols, w_cols = [], []
    for _ in range(TOP_K):
        m = jnp.max(tmp, axis=-1, keepdims=True)
        i = jnp.min(jnp.where(tmp == m, lane_f, jnp.float32(N_EXPERTS)),
                    axis=-1, keepdims=True)
        onehot = lane_f == i
        w_cols.append(jnp.sum(jnp.where(onehot, s, 0.0), axis=-1,
                              keepdims=True))
        idx_cols.append(i)
        tmp = jnp.where(onehot, neg, tmp)

    idx = jnp.concatenate(idx_cols, axis=1).astype(jnp.int32)
    wgt = jnp.concatenate(w_cols, axis=1)
    denom = jnp.sum(wgt, axis=-1, keepdims=True) + 1e-20
    wgt = wgt * (SCALE / denom)

    idx_ref[...] = idx
    wgt_ref[...] = wgt


@functools.partial(jax.jit, static_argnames=())
def kernel(hidden_states, weight, e_score_correction_bias):
    bsz, seq_len, h = hidden_states.shape
    n = bsz * seq_len
    x = hidden_states.reshape(n, h).astype(jnp.float32)
    w = weight.astype(jnp.float32)
    b = e_score_correction_bias.astype(jnp.float32).reshape(1, N_EXPERTS)

    grid = (n // TILE,)
    idx, wgt = pl.pallas_call(
        _gate_kernel,
        grid=grid,
        in_specs=[
            pl.BlockSpec((TILE, h), lambda i: (i, 0)),
            pl.BlockSpec((N_EXPERTS, h), lambda i: (0, 0)),
            pl.BlockSpec((1, N_EXPERTS), lambda i: (0, 0)),
        ],
        out_specs=[
            pl.BlockSpec((TILE, TOP_K), lambda i: (i, 0)),
            pl.BlockSpec((TILE, TOP_K), lambda i: (i, 0)),
        ],
        out_shape=[
            jax.ShapeDtypeStruct((n, TOP_K), jnp.int32),
            jax.ShapeDtypeStruct((n, TOP_K), jnp.float32),
        ],
        compiler_params=pltpu.CompilerParams(
            dimension_semantics=("parallel",),
        ),
    )(x, w, b)
    return idx, wgt


# speedup vs baseline: 2.9777x; 2.9777x over previous
import functools

import jax
import jax.numpy as jnp
from jax.experimental import pallas as pl
from jax.experimental.pallas import tpu as pltpu

TOP_K = 8
N_EXPERTS = 64
N_GROUP = 8
GROUP_SIZE = N_EXPERTS // N_GROUP
TOPK_GROUP = 4
SCALE = 2.5

TILE = 1024


def _gate_kernel(x_ref, w_ref, b_ref, idx_ref, wgt_ref):
    x = x_ref[...]
    w = w_ref[...]
    logits = jax.lax.dot_general(
        w, x, (((1,), (1,)), ((), ())),
        preferred_element_type=jnp.float32,
    )
    s = jax.nn.sigmoid(logits)
    sc = s + b_ref[...]

    t = x.shape[0]
    row = jax.lax.broadcasted_iota(jnp.int32, (N_EXPERTS, t), 0)
    row_f = row.astype(jnp.float32)
    gid = row // GROUP_SIZE
    rowm = row % GROUP_SIZE
    neg = jnp.float32(-jnp.inf)

    a = sc
    b = jnp.full_like(sc, neg)
    for d in (1, 2, 4):
        ar = jnp.roll(a, -d, axis=0)
        br = jnp.roll(b, -d, axis=0)
        valid = rowm < (GROUP_SIZE - d)
        ar = jnp.where(valid, ar, neg)
        br = jnp.where(valid, br, neg)
        b = jnp.maximum(jnp.maximum(b, br), jnp.minimum(a, ar))
        a = jnp.maximum(a, ar)
    g = a + b
    for d in (1, 2, 4):
        g = jnp.where(rowm >= d, jnp.roll(g, d, axis=0), g)

    rank = jnp.zeros_like(row)
    for k in range(1, N_GROUP):
        other = jnp.roll(g, -GROUP_SIZE * k, axis=0)
        og_lt = gid >= (N_GROUP - k)
        beats = (other > g) | ((other == g) & og_lt)
        rank = rank + beats.astype(jnp.int32)
    tmp = jnp.where(rank < TOPK_GROUP, sc, neg)

    idx_rows, w_rows = [], []
    for _ in range(TOP_K):
        m = jnp.max(tmp, axis=0, keepdims=True)
        i = jnp.min(jnp.where(tmp == m, row_f, jnp.float32(N_EXPERTS)),
                    axis=0, keepdims=True)
        onehot = row_f == i
        w_rows.append(jnp.sum(jnp.where(onehot, s, 0.0), axis=0,
                              keepdims=True))
        idx_rows.append(i)
        tmp = jnp.where(onehot, neg, tmp)

    idx = jnp.concatenate(idx_rows, axis=0).astype(jnp.int32)
    wgt = jnp.concatenate(w_rows, axis=0)
    denom = jnp.sum(wgt, axis=0, keepdims=True) + 1e-20
    wgt = wgt * (SCALE / denom)

    idx_ref[...] = idx
    wgt_ref[...] = wgt


@functools.partial(jax.jit, static_argnames=())
def kernel(hidden_states, weight, e_score_correction_bias):
    bsz, seq_len, h = hidden_states.shape
    n = bsz * seq_len
    x = hidden_states.reshape(n, h).astype(jnp.float32)
    w = weight.astype(jnp.float32)
    b = e_score_correction_bias.astype(jnp.float32).reshape(N_EXPERTS, 1)

    grid = (n // TILE,)
    idx_t, wgt_t = pl.pallas_call(
        _gate_kernel,
        grid=grid,
        in_specs=[
            pl.BlockSpec((TILE, h), lambda i: (i, 0)),
            pl.BlockSpec((N_EXPERTS, h), lambda i: (0, 0)),
            pl.BlockSpec((N_EXPERTS, 1), lambda i: (0, 0)),
        ],
        out_specs=[
            pl.BlockSpec((TOP_K, TILE), lambda i: (0, i)),
            pl.BlockSpec((TOP_K, TILE), lambda i: (0, i)),
        ],
        out_shape=[
            jax.ShapeDtypeStruct((TOP_K, n), jnp.int32),
            jax.ShapeDtypeStruct((TOP_K, n), jnp.float32),
        ],
        compiler_params=pltpu.CompilerParams(
            dimension_semantics=("parallel",),
        ),
    )(x, w, b)
    return idx_t.T, wgt_t.T


# speedup vs baseline: 3.1991x; 1.0744x over previous
import functools

import jax
import jax.numpy as jnp
from jax.experimental import pallas as pl
from jax.experimental.pallas import tpu as pltpu

TOP_K = 8
N_EXPERTS = 64
N_GROUP = 8
GROUP_SIZE = N_EXPERTS // N_GROUP
TOPK_GROUP = 4
SCALE = 2.5

TILE = 2048


def _gate_kernel(x_ref, w_ref, b_ref, idx_ref, wgt_ref):
    x = x_ref[...]
    w = w_ref[...]
    logits = jax.lax.dot_general(
        w, x, (((1,), (1,)), ((), ())),
        preferred_element_type=jnp.float32,
    )
    s = jax.nn.sigmoid(logits)
    sc = s + b_ref[...]

    t = x.shape[0]
    row = jax.lax.broadcasted_iota(jnp.int32, (N_EXPERTS, t), 0)
    row_f = row.astype(jnp.float32)
    gid = row // GROUP_SIZE
    rowm = row % GROUP_SIZE
    neg = jnp.float32(-jnp.inf)

    a = sc
    b = jnp.full_like(sc, neg)
    for d in (1, 2, 4):
        ar = jnp.roll(a, -d, axis=0)
        br = jnp.roll(b, -d, axis=0)
        valid = rowm < (GROUP_SIZE - d)
        ar = jnp.where(valid, ar, neg)
        br = jnp.where(valid, br, neg)
        b = jnp.maximum(jnp.maximum(b, br), jnp.minimum(a, ar))
        a = jnp.maximum(a, ar)
    g = a + b
    for d in (1, 2, 4):
        g = jnp.where(rowm >= d, jnp.roll(g, d, axis=0), g)

    rank = jnp.zeros_like(row)
    for k in range(1, N_GROUP):
        other = jnp.roll(g, -GROUP_SIZE * k, axis=0)
        og_lt = gid >= (N_GROUP - k)
        beats = (other > g) | ((other == g) & og_lt)
        rank = rank + beats.astype(jnp.int32)
    tmp = jnp.where(rank < TOPK_GROUP, sc, neg)

    idx_rows, w_rows = [], []
    for _ in range(TOP_K):
        m = jnp.max(tmp, axis=0, keepdims=True)
        i = jnp.min(jnp.where(tmp == m, row_f, jnp.float32(N_EXPERTS)),
                    axis=0, keepdims=True)
        onehot = row_f == i
        w_rows.append(jnp.sum(jnp.where(onehot, s, 0.0), axis=0,
                              keepdims=True))
        idx_rows.append(i)
        tmp = jnp.where(onehot, neg, tmp)

    idx = jnp.concatenate(idx_rows, axis=0).astype(jnp.int32)
    wgt = jnp.concatenate(w_rows, axis=0)
    denom = jnp.sum(wgt, axis=0, keepdims=True) + 1e-20
    wgt = wgt * (SCALE / denom)

    idx_ref[...] = idx
    wgt_ref[...] = wgt


@functools.partial(jax.jit, static_argnames=())
def kernel(hidden_states, weight, e_score_correction_bias):
    bsz, seq_len, h = hidden_states.shape
    n = bsz * seq_len
    x = hidden_states.reshape(n, h).astype(jnp.float32)
    w = weight.astype(jnp.float32)
    b = e_score_correction_bias.astype(jnp.float32).reshape(N_EXPERTS, 1)

    grid = (n // TILE,)
    idx_t, wgt_t = pl.pallas_call(
        _gate_kernel,
        grid=grid,
        in_specs=[
            pl.BlockSpec((TILE, h), lambda i: (i, 0)),
            pl.BlockSpec((N_EXPERTS, h), lambda i: (0, 0)),
            pl.BlockSpec((N_EXPERTS, 1), lambda i: (0, 0)),
        ],
        out_specs=[
            pl.BlockSpec((TOP_K, TILE), lambda i: (0, i)),
            pl.BlockSpec((TOP_K, TILE), lambda i: (0, i)),
        ],
        out_shape=[
            jax.ShapeDtypeStruct((TOP_K, n), jnp.int32),
            jax.ShapeDtypeStruct((TOP_K, n), jnp.float32),
        ],
        compiler_params=pltpu.CompilerParams(
            dimension_semantics=("parallel",),
        ),
    )(x, w, b)
    return idx_t.T, wgt_t.T


# speedup vs baseline: 3.2904x; 1.0285x over previous
import functools

import jax
import jax.numpy as jnp
from jax.experimental import pallas as pl
from jax.experimental.pallas import tpu as pltpu

TOP_K = 8
N_EXPERTS = 64
N_GROUP = 8
GROUP_SIZE = N_EXPERTS // N_GROUP
TOPK_GROUP = 4
SCALE = 2.5

TILE = 2048


def _gate_kernel(x_ref, w_ref, b_ref, idx_ref, wgt_ref):
    x = x_ref[...]
    w = w_ref[...]
    logits = jax.lax.dot_general(
        w, x, (((1,), (1,)), ((), ())),
        preferred_element_type=jnp.float32,
    )
    s = jax.nn.sigmoid(logits)
    sc = s + b_ref[...]

    t = x.shape[0]
    row = jax.lax.broadcasted_iota(jnp.int32, (N_EXPERTS, t), 0)
    row_f = row.astype(jnp.float32)
    gid = row // GROUP_SIZE
    rowm = row % GROUP_SIZE
    neg = jnp.float32(-jnp.inf)

    sc3 = sc.reshape(N_GROUP, GROUP_SIZE, t)
    rowm3 = jax.lax.broadcasted_iota(jnp.int32, (N_GROUP, GROUP_SIZE, t), 1)
    a = sc3
    b = jnp.full_like(sc3, neg)
    for d in (1, 2, 4):
        ar = jnp.roll(a, -d, axis=1)
        br = jnp.roll(b, -d, axis=1)
        valid = rowm3 < (GROUP_SIZE - d)
        ar = jnp.where(valid, ar, neg)
        br = jnp.where(valid, br, neg)
        b = jnp.maximum(jnp.maximum(b, br), jnp.minimum(a, ar))
        a = jnp.maximum(a, ar)
    g8 = (a + b)[:, 0, :]

    gidx = jax.lax.broadcasted_iota(jnp.int32, (N_GROUP, t), 0)
    rank = jnp.zeros_like(gidx)
    for k in range(1, N_GROUP):
        other = jnp.roll(g8, -k, axis=0)
        og_lt = gidx >= (N_GROUP - k)
        beats = (other > g8) | ((other == g8) & og_lt)
        rank = rank + beats.astype(jnp.int32)
    keep8 = rank < TOPK_GROUP
    tmp = jnp.where(keep8[:, None, :], sc3, neg).reshape(N_EXPERTS, t)

    idx_rows, w_rows = [], []
    for _ in range(TOP_K):
        m = jnp.max(tmp, axis=0, keepdims=True)
        i = jnp.min(jnp.where(tmp == m, row_f, jnp.float32(N_EXPERTS)),
                    axis=0, keepdims=True)
        onehot = row_f == i
        w_rows.append(jnp.sum(jnp.where(onehot, s, 0.0), axis=0,
                              keepdims=True))
        idx_rows.append(i)
        tmp = jnp.where(onehot, neg, tmp)

    idx = jnp.concatenate(idx_rows, axis=0).astype(jnp.int32)
    wgt = jnp.concatenate(w_rows, axis=0)
    denom = jnp.sum(wgt, axis=0, keepdims=True) + 1e-20
    wgt = wgt * (SCALE / denom)

    idx_ref[...] = idx
    wgt_ref[...] = wgt


@functools.partial(jax.jit, static_argnames=())
def kernel(hidden_states, weight, e_score_correction_bias):
    bsz, seq_len, h = hidden_states.shape
    n = bsz * seq_len
    x = hidden_states.reshape(n, h).astype(jnp.float32)
    w = weight.astype(jnp.float32)
    b = e_score_correction_bias.astype(jnp.float32).reshape(N_EXPERTS, 1)

    grid = (n // TILE,)
    idx_t, wgt_t = pl.pallas_call(
        _gate_kernel,
        grid=grid,
        in_specs=[
            pl.BlockSpec((TILE, h), lambda i: (i, 0)),
            pl.BlockSpec((N_EXPERTS, h), lambda i: (0, 0)),
            pl.BlockSpec((N_EXPERTS, 1), lambda i: (0, 0)),
        ],
        out_specs=[
            pl.BlockSpec((TOP_K, TILE), lambda i: (0, i)),
            pl.BlockSpec((TOP_K, TILE), lambda i: (0, i)),
        ],
        out_shape=[
            jax.ShapeDtypeStruct((TOP_K, n), jnp.int32),
            jax.ShapeDtypeStruct((TOP_K, n), jnp.float32),
        ],
        compiler_params=pltpu.CompilerParams(
            dimension_semantics=("parallel",),
        ),
    )(x, w, b)
    return idx_t.T, wgt_t.T


# speedup vs baseline: 3.3703x; 1.0243x over previous
import functools

import jax
import jax.numpy as jnp
from jax.experimental import pallas as pl
from jax.experimental.pallas import tpu as pltpu

TOP_K = 8
N_EXPERTS = 64
N_GROUP = 8
GROUP_SIZE = N_EXPERTS // N_GROUP
TOPK_GROUP = 4
SCALE = 2.5

TILE = 2048


def _gate_kernel(x_ref, w_ref, b_ref, idx_ref, wgt_ref):
    x = x_ref[...]
    w = w_ref[...]
    logits = jax.lax.dot_general(
        w, x, (((1,), (1,)), ((), ())),
        preferred_element_type=jnp.float32,
    )
    s = jax.nn.sigmoid(logits)
    sc = s + b_ref[...]

    t = x.shape[0]
    row = jax.lax.broadcasted_iota(jnp.int32, (N_EXPERTS, t), 0)
    row_f = row.astype(jnp.float32)
    neg = jnp.float32(-jnp.inf)

    sc3 = sc.reshape(N_GROUP, GROUP_SIZE, t)
    rowm3 = jax.lax.broadcasted_iota(jnp.int32, (N_GROUP, GROUP_SIZE, t), 1)
    a = sc3
    b = jnp.full_like(sc3, neg)
    for d in (1, 2, 4):
        ar = jnp.roll(a, -d, axis=1)
        br = jnp.roll(b, -d, axis=1)
        valid = rowm3 < (GROUP_SIZE - d)
        ar = jnp.where(valid, ar, neg)
        br = jnp.where(valid, br, neg)
        b = jnp.maximum(jnp.maximum(b, br), jnp.minimum(a, ar))
        a = jnp.maximum(a, ar)
    g8 = (a + b)[:, 0, :]

    gidx = jax.lax.broadcasted_iota(jnp.int32, (N_GROUP, t), 0)
    rank = jnp.zeros_like(gidx)
    for k in range(1, N_GROUP):
        other = jnp.roll(g8, -k, axis=0)
        og_lt = gidx >= (N_GROUP - k)
        beats = (other > g8) | ((other == g8) & og_lt)
        rank = rank + beats.astype(jnp.int32)
    keep8 = rank < TOPK_GROUP
    tmp = jnp.where(keep8[:, None, :], sc3, neg).reshape(N_EXPERTS, t)

    idx_rows, w_rows = [], []
    for _ in range(TOP_K):
        m = jnp.max(tmp, axis=0, keepdims=True)
        i = jnp.min(jnp.where(tmp == m, row_f, jnp.float32(N_EXPERTS)),
                    axis=0, keepdims=True)
        onehot = row_f == i
        w_rows.append(jnp.sum(jnp.where(onehot, s, 0.0), axis=0,
                              keepdims=True))
        idx_rows.append(i)
        tmp = jnp.where(onehot, neg, tmp)

    idx = jnp.concatenate(idx_rows, axis=0).astype(jnp.int32)
    wgt = jnp.concatenate(w_rows, axis=0)
    denom = jnp.sum(wgt, axis=0, keepdims=True) + 1e-20
    wgt = wgt * (SCALE / denom)

    idx_ref[...] = idx
    wgt_ref[...] = wgt


@functools.partial(jax.jit, static_argnames=())
def kernel(hidden_states, weight, e_score_correction_bias):
    bsz, seq_len, h = hidden_states.shape
    n = bsz * seq_len
    x = hidden_states.reshape(n, h).astype(jnp.float32)
    w = weight.astype(jnp.float32)
    b = e_score_correction_bias.astype(jnp.float32).reshape(N_EXPERTS, 1)

    grid = (n // TILE,)
    idx_t, wgt_t = pl.pallas_call(
        _gate_kernel,
        grid=grid,
        in_specs=[
            pl.BlockSpec((TILE, h), lambda i: (i, 0)),
            pl.BlockSpec((N_EXPERTS, h), lambda i: (0, 0)),
            pl.BlockSpec((N_EXPERTS, 1), lambda i: (0, 0)),
        ],
        out_specs=[
            pl.BlockSpec((TOP_K, TILE), lambda i: (0, i)),
            pl.BlockSpec((TOP_K, TILE), lambda i: (0, i)),
        ],
        out_shape=[
            jax.ShapeDtypeStruct((TOP_K, n), jnp.int32),
            jax.ShapeDtypeStruct((TOP_K, n), jnp.float32),
        ],
        compiler_params=pltpu.CompilerParams(
            dimension_semantics=("parallel",),
        ),
    )(x, w, b)
    return idx_t.T, wgt_t.T
